# baseline (speedup 1.0000x reference)
import jax
from jax.experimental import pallas as pl
from jax.experimental.pallas import tpu as pltpu


def _dma_body(x_ref, o_ref, sem):
    cp = pltpu.make_async_copy(x_ref, o_ref, sem)
    cp.start()
    cp.wait()


def kernel(x, center):
    del center
    return pl.pallas_call(
        _dma_body,
        in_specs=[pl.BlockSpec(memory_space=pltpu.MemorySpace.HBM)],
        out_specs=pl.BlockSpec(memory_space=pltpu.MemorySpace.HBM),
        out_shape=jax.ShapeDtypeStruct(x.shape, x.dtype),
        scratch_shapes=[pltpu.SemaphoreType.DMA],
    )(x)


# speedup vs baseline: 465.6851x; 465.6851x over previous
import jax
from jax.experimental import pallas as pl
from jax.experimental.pallas import tpu as pltpu


def _copy_body(x_ref, o_ref):
    o_ref[...] = x_ref[...]


def kernel(x, center):
    del center
    xt = x.transpose(2, 3, 0, 1)
    yt = pl.pallas_call(
        _copy_body,
        grid=(14,),
        in_specs=[pl.BlockSpec((1, 14, 256, 768), lambda i: (i, 0, 0, 0))],
        out_specs=pl.BlockSpec((1, 14, 256, 768), lambda i: (i, 0, 0, 0)),
        out_shape=jax.ShapeDtypeStruct((14, 14, 256, 768), x.dtype),
        compiler_params=pltpu.CompilerParams(
            dimension_semantics=("parallel",),
        ),
    )(xt)
    return yt.transpose(2, 3, 0, 1)
